# Initial kernel scaffold; baseline (speedup 1.0000x reference)
import jax
import jax.numpy as jnp
from jax.experimental import pallas as pl


def kernel(unary, image):
    raise NotImplementedError("write your pallas kernel here")


# speedup vs baseline: 1.9269x; 1.9269x over previous
import jax
import jax.numpy as jnp
from jax.experimental import pallas as pl
from jax.experimental.pallas import tpu as pltpu

_H, _W, _C = 64, 64, 21
_N = _H * _W
_BLK = 128
_IBLK = 512
_NBLK = _N // _BLK
_NUM_ITERS = 5
_BI_W = 10.0
_SP_W = 3.0


def _softmax_rows(z):
    z = z - jnp.max(z, axis=-1, keepdims=True)
    e = jnp.exp(z)
    return e / jnp.sum(e, axis=-1, keepdims=True)


def _crf_body(abr_ref, abc_ref, asr_ref, asc_ref, u_ref, out_ref,
              kc_ref, qa_ref, qb_ref):
    abc = abc_ref[...]
    asc = asc_ref[...]

    def _build(rb, carry):
        lo = rb * _BLK
        abr = abr_ref[pl.ds(lo, _BLK), :]
        asr = asr_ref[pl.ds(lo, _BLK), :]
        ffb = jnp.dot(abr[:, 0:5], abc[0:5, :],
                      preferred_element_type=jnp.float32)
        ffs = jnp.dot(asr[:, 0:2], asc[0:2, :],
                      preferred_element_type=jnp.float32)
        arg_b = abr[:, 5:6] + abc[6:7, :] + ffb
        arg_s = asr[:, 2:3] + asc[3:4, :] + ffs
        kb = jnp.exp(jnp.minimum(arg_b, 0.0))
        ks = jnp.exp(jnp.minimum(arg_s, 0.0))
        nb = jnp.maximum(jnp.sum(kb, axis=1, keepdims=True) - 1.0, 1e-20)
        ns = jnp.maximum(jnp.sum(ks, axis=1, keepdims=True) - 1.0, 1e-20)
        wb = _BI_W / nb
        ws = _SP_W / ns
        kc = kb * wb + ks * ws
        rows = lo + jax.lax.broadcasted_iota(jnp.int32, (_BLK, _N), 0)
        cols = jax.lax.broadcasted_iota(jnp.int32, (_BLK, _N), 1)
        kc = kc - jnp.where(rows == cols, wb + ws, 0.0)
        kc_ref[pl.ds(lo, _BLK), :] = kc.astype(jnp.bfloat16)
        return carry

    jax.lax.fori_loop(0, _NBLK, _build, 0)

    qa_ref[...] = _softmax_rows(-u_ref[...])
    bufs = [qa_ref, qb_ref]
    for t in range(_NUM_ITERS):
        src, dst = bufs[t % 2], bufs[(t + 1) % 2]
        q_cur = src[...].astype(jnp.bfloat16)

        def _update(rb, carry, q_cur=q_cur, dst=dst):
            lo = rb * _IBLK
            msg = jnp.dot(kc_ref[pl.ds(lo, _IBLK), :], q_cur,
                          preferred_element_type=jnp.float32)
            dst[pl.ds(lo, _IBLK), :] = _softmax_rows(
                msg - u_ref[pl.ds(lo, _IBLK), :])
            return carry

        jax.lax.fori_loop(0, _N // _IBLK, _update, 0)
    out_ref[...] = bufs[_NUM_ITERS % 2][...]


def kernel(unary, image):
    ys, xs = jnp.meshgrid(jnp.arange(_H, dtype=jnp.float32),
                          jnp.arange(_W, dtype=jnp.float32), indexing='ij')
    xs = xs.reshape(-1)
    ys = ys.reshape(-1)
    rgb = image.reshape(_N, 3) * 255.0
    bi = jnp.concatenate([(xs / 80.0)[:, None], (ys / 80.0)[:, None],
                          rgb / 13.0], axis=1)
    sp = jnp.stack([xs / 3.0, ys / 3.0], axis=1)
    msq_b = -0.5 * jnp.sum(bi * bi, axis=1, keepdims=True)
    msq_s = -0.5 * jnp.sum(sp * sp, axis=1, keepdims=True)
    one = jnp.ones((_N, 1), jnp.float32)
    zero = jnp.zeros((_N, 1), jnp.float32)
    ab_row = jnp.concatenate([bi, msq_b, one, zero], axis=1)
    ab_col = jnp.concatenate([bi, one, msq_b, zero], axis=1).T
    as_row = jnp.concatenate([sp, msq_s, one] + [zero] * 4, axis=1)
    as_col = jnp.concatenate([sp, one, msq_s] + [zero] * 4, axis=1).T
    u = unary.reshape(_N, _C)

    q = pl.pallas_call(
        _crf_body,
        out_shape=jax.ShapeDtypeStruct((_N, _C), jnp.float32),
        scratch_shapes=[
            pltpu.VMEM((_N, _N), jnp.bfloat16),
            pltpu.VMEM((_N, _C), jnp.float32),
            pltpu.VMEM((_N, _C), jnp.float32),
        ],
    )(ab_row, ab_col, as_row, as_col, u)
    return q.reshape(_H, _W, _C)
